# baseline (speedup 1.0000x reference)
import jax
import jax.numpy as jnp
from jax import lax
from jax.experimental import pallas as pl
from jax.experimental.pallas import tpu as pltpu
from jax.experimental.pallas import tpu_sc as plsc

R, V, B = 64, 1_000_000, 16384
NC, NS = 2, 16
RPC = R // NC
SEG = B // NS
H = V // 2
NH = RPC * 2
L = 16


def _body(data_hbm, idx_hbm, out_hbm,
          idx_v, idxa_v, idxb_v, ga_v, gb_v, out_v, sh0, sh1, sh2,
          rsem0, rsem1, rsem2, gsem, osem):
    cid = lax.axis_index("c")
    sid = lax.axis_index("s")
    row_sh = (sh0, sh1, sh2)
    rsems = (rsem0, rsem1, rsem2)
    pltpu.sync_copy(idx_hbm.at[pl.ds(sid * SEG, SEG)], idx_v)

    def prep(k, carry):
        v = idx_v[pl.ds(k * L, L)]
        idxa_v[pl.ds(k * L, L)] = jnp.minimum(v, H - 1)
        idxb_v[pl.ds(k * L, L)] = jnp.maximum(v - H, 0)
        return carry

    lax.fori_loop(0, SEG // L, prep, 0)

    def src(h):
        return data_hbm.at[cid * RPC + h // 2].at[h % 2]

    def stage(h):
        @pl.when(sid == h % 2)
        def _():
            pltpu.async_copy(src(h), row_sh[h % 3], rsems[h % 3])

    def wait_stage(h):
        @pl.when(sid == h % 2)
        def _():
            pltpu.make_async_copy(src(h), row_sh[h % 3],
                                  rsems[h % 3]).wait()

    stage(0)
    stage(1)
    ocp = [None, None]
    for h in range(NH):
        row_k, part = h // 2, h % 2
        wait_stage(h)
        plsc.subcore_barrier()
        if h + 2 < NH:
            stage(h + 2)
        gl_v = ga_v if part == 0 else gb_v
        il_v = idxa_v if part == 0 else idxb_v
        pltpu.async_copy(row_sh[h % 3].at[il_v], gl_v, gsem).wait()
        if part == 1:
            if ocp[row_k % 2] is not None:
                ocp[row_k % 2].wait()

            def merge(k, carry):
                v = idx_v[pl.ds(k * L, L)]
                a = ga_v[pl.ds(k * L, L)]
                b = gb_v[pl.ds(k * L, L)]
                out_v[row_k % 2, pl.ds(k * L, L)] = jnp.where(v < H, a, b)
                return carry

            lax.fori_loop(0, SEG // L, merge, 0)
            r = cid * RPC + row_k
            ocp[row_k % 2] = pltpu.async_copy(
                out_v.at[row_k % 2],
                out_hbm.at[r].at[pl.ds(sid * SEG, SEG)], osem)
    for o in ocp:
        o.wait()


def kernel(data, indices):
    data3 = data.reshape(R, 2, H)
    mesh = plsc.VectorSubcoreMesh(core_axis_name="c", subcore_axis_name="s")
    gather = pl.kernel(
        _body,
        mesh=mesh,
        out_type=jax.ShapeDtypeStruct((R, B), jnp.float32),
        scratch_types=[
            pltpu.VMEM((SEG,), jnp.int32),
            pltpu.VMEM((SEG,), jnp.int32),
            pltpu.VMEM((SEG,), jnp.int32),
            pltpu.VMEM((SEG,), jnp.float32),
            pltpu.VMEM((SEG,), jnp.float32),
            pltpu.VMEM((2, SEG), jnp.float32),
            pltpu.VMEM_SHARED((H,), jnp.float32),
            pltpu.VMEM_SHARED((H,), jnp.float32),
            pltpu.VMEM_SHARED((H,), jnp.float32),
            pltpu.SemaphoreType.DMA,
            pltpu.SemaphoreType.DMA,
            pltpu.SemaphoreType.DMA,
            pltpu.SemaphoreType.DMA,
            pltpu.SemaphoreType.DMA,
        ],
    )
    return gather(data3, indices)


# speedup vs baseline: 1.1485x; 1.1485x over previous
import jax
import jax.numpy as jnp
from jax import lax
from jax.experimental import pallas as pl
from jax.experimental.pallas import tpu as pltpu
from jax.experimental.pallas import tpu_sc as plsc

R, V, B = 64, 1_000_000, 16384
NC, NS = 2, 16
RPC = R // NC
SEG = B // NS
H = V // 2
NH = RPC * 2
L = 16


def _body(data_hbm, idx_hbm, out_hbm,
          idx_v, idxa_v, ga_v, gb_v, out_v, sh0, sh1, sh2,
          rsem0, rsem1, rsem2, gsem, osem):
    cid = lax.axis_index("c")
    sid = lax.axis_index("s")
    row_sh = (sh0, sh1, sh2)
    rsems = (rsem0, rsem1, rsem2)
    pltpu.sync_copy(idx_hbm.at[pl.ds(sid * SEG, SEG)], idx_v)

    def prep(k, carry):
        v = idx_v[pl.ds(k * L, L)]
        idxa_v[pl.ds(k * L, L)] = jnp.where(v < H, v, v - H)
        return carry

    lax.fori_loop(0, SEG // L, prep, 0)

    def src(h):
        return data_hbm.at[cid * RPC + h // 2].at[h % 2]

    def stage(h):
        @pl.when(sid == h % 2)
        def _():
            pltpu.async_copy(src(h), row_sh[h % 3], rsems[h % 3])

    def wait_stage(h):
        @pl.when(sid == h % 2)
        def _():
            pltpu.make_async_copy(src(h), row_sh[h % 3],
                                  rsems[h % 3]).wait()

    stage(0)
    stage(1)
    ocp = [None, None]
    for h in range(NH):
        row_k, part = h // 2, h % 2
        wait_stage(h)
        plsc.subcore_barrier()
        if h + 2 < NH:
            stage(h + 2)
        gl_v = ga_v if part == 0 else gb_v
        pltpu.async_copy(row_sh[h % 3].at[idxa_v], gl_v, gsem).wait()
        if part == 1:
            if ocp[row_k % 2] is not None:
                ocp[row_k % 2].wait()

            def merge(k, carry):
                v = idx_v[pl.ds(k * L, L)]
                a = ga_v[pl.ds(k * L, L)]
                b = gb_v[pl.ds(k * L, L)]
                out_v[row_k % 2, pl.ds(k * L, L)] = jnp.where(v < H, a, b)
                return carry

            lax.fori_loop(0, SEG // L, merge, 0)
            r = cid * RPC + row_k
            ocp[row_k % 2] = pltpu.async_copy(
                out_v.at[row_k % 2],
                out_hbm.at[r].at[pl.ds(sid * SEG, SEG)], osem)
    for o in ocp:
        o.wait()


def kernel(data, indices):
    data3 = data.reshape(R, 2, H)
    mesh = plsc.VectorSubcoreMesh(core_axis_name="c", subcore_axis_name="s")
    gather = pl.kernel(
        _body,
        mesh=mesh,
        out_type=jax.ShapeDtypeStruct((R, B), jnp.float32),
        scratch_types=[
            pltpu.VMEM((SEG,), jnp.int32),
            pltpu.VMEM((SEG,), jnp.int32),
            pltpu.VMEM((SEG,), jnp.float32),
            pltpu.VMEM((SEG,), jnp.float32),
            pltpu.VMEM((2, SEG), jnp.float32),
            pltpu.VMEM_SHARED((H,), jnp.float32),
            pltpu.VMEM_SHARED((H,), jnp.float32),
            pltpu.VMEM_SHARED((H,), jnp.float32),
            pltpu.SemaphoreType.DMA,
            pltpu.SemaphoreType.DMA,
            pltpu.SemaphoreType.DMA,
            pltpu.SemaphoreType.DMA,
            pltpu.SemaphoreType.DMA,
        ],
    )
    return gather(data3, indices)


# speedup vs baseline: 7.6093x; 6.6253x over previous
import jax
import jax.numpy as jnp
from jax import lax
from jax.experimental import pallas as pl
from jax.experimental.pallas import tpu as pltpu
from jax.experimental.pallas import tpu_sc as plsc

R, V, B = 64, 1_000_000, 16384
NC, NS = 2, 16
RPC = R // NC
SEG = B // NS
W0 = 499968
W1 = V - W0
NH = RPC * 2
L = 16


def _body(data_hbm, idx_hbm, out_hbm,
          idx_v, idxa_v, ga_v, gb_v, out_v, sh0, sh1, sh2,
          rsem0, rsem1, rsem2, gsem, osem):
    cid = lax.axis_index("c")
    sid = lax.axis_index("s")
    row_sh = (sh0, sh1, sh2)
    rsems = (rsem0, rsem1, rsem2)
    pltpu.sync_copy(idx_hbm.at[pl.ds(sid * SEG, SEG)], idx_v)

    def prep(k, carry):
        v = idx_v[pl.ds(k * L, L)]
        idxa_v[pl.ds(k * L, L)] = jnp.where(v < W0, v, v - W0)
        return carry

    lax.fori_loop(0, SEG // L, prep, 0)

    def src(h):
        off, n = (0, W0) if h % 2 == 0 else (W0, W1)
        return data_hbm.at[cid * RPC + h // 2].at[pl.ds(off, n)]

    def dst(h):
        n = W0 if h % 2 == 0 else W1
        return row_sh[h % 3].at[pl.ds(0, n)]

    def stage(h):
        @pl.when(sid == h % 2)
        def _():
            pltpu.async_copy(src(h), dst(h), rsems[h % 3])

    def wait_stage(h):
        @pl.when(sid == h % 2)
        def _():
            pltpu.make_async_copy(src(h), dst(h), rsems[h % 3]).wait()

    stage(0)
    stage(1)
    ocp = [None, None]
    for h in range(NH):
        row_k, part = h // 2, h % 2
        wait_stage(h)
        plsc.subcore_barrier()
        if h + 2 < NH:
            stage(h + 2)
        gl_v = ga_v if part == 0 else gb_v
        pltpu.async_copy(row_sh[h % 3].at[idxa_v], gl_v, gsem).wait()
        if part == 1:
            if ocp[row_k % 2] is not None:
                ocp[row_k % 2].wait()

            def merge(k, carry):
                v = idx_v[pl.ds(k * L, L)]
                a = ga_v[pl.ds(k * L, L)]
                b = gb_v[pl.ds(k * L, L)]
                out_v[row_k % 2, pl.ds(k * L, L)] = jnp.where(v < W0, a, b)
                return carry

            lax.fori_loop(0, SEG // L, merge, 0)
            r = cid * RPC + row_k
            ocp[row_k % 2] = pltpu.async_copy(
                out_v.at[row_k % 2],
                out_hbm.at[r].at[pl.ds(sid * SEG, SEG)], osem)
    for o in ocp:
        o.wait()


def kernel(data, indices):
    mesh = plsc.VectorSubcoreMesh(core_axis_name="c", subcore_axis_name="s")
    gather = pl.kernel(
        _body,
        mesh=mesh,
        out_type=jax.ShapeDtypeStruct((R, B), jnp.float32),
        scratch_types=[
            pltpu.VMEM((SEG,), jnp.int32),
            pltpu.VMEM((SEG,), jnp.int32),
            pltpu.VMEM((SEG,), jnp.float32),
            pltpu.VMEM((SEG,), jnp.float32),
            pltpu.VMEM((2, SEG), jnp.float32),
            pltpu.VMEM_SHARED((W1,), jnp.float32),
            pltpu.VMEM_SHARED((W1,), jnp.float32),
            pltpu.VMEM_SHARED((W1,), jnp.float32),
            pltpu.SemaphoreType.DMA,
            pltpu.SemaphoreType.DMA,
            pltpu.SemaphoreType.DMA,
            pltpu.SemaphoreType.DMA,
            pltpu.SemaphoreType.DMA,
        ],
    )
    return gather(data, indices)


# speedup vs baseline: 7.6110x; 1.0002x over previous
import jax
import jax.numpy as jnp
from jax import lax
from jax.experimental import pallas as pl
from jax.experimental.pallas import tpu as pltpu
from jax.experimental.pallas import tpu_sc as plsc

R, V, B = 64, 1_000_000, 16384
NC, NS = 2, 16
RPC = R // NC
SEG = B // NS
W0 = 499968
W1 = V - W0
NH = RPC * 2
L = 16


def _body(data_hbm, idx_hbm, out_hbm,
          idx_v, idxa_v, ga_v, gb_v, out_v, sh0, sh1, sh2,
          rsem0, rsem1, rsem2, gsem, osem):
    cid = lax.axis_index("c")
    sid = lax.axis_index("s")
    row_sh = (sh0, sh1, sh2)
    rsems = (rsem0, rsem1, rsem2)
    pltpu.sync_copy(idx_hbm.at[pl.ds(sid * SEG, SEG)], idx_v)

    def prep(k, carry):
        v = idx_v[pl.ds(k * L, L)]
        idxa_v[pl.ds(k * L, L)] = jnp.where(v < W0, v, v - W0)
        return carry

    lax.fori_loop(0, SEG // L, prep, 0)

    def src(h):
        off, n = (0, W0) if h % 2 == 0 else (W0, W1)
        return data_hbm.at[cid * RPC + h // 2].at[pl.ds(off, n)]

    def dst(h):
        n = W0 if h % 2 == 0 else W1
        return row_sh[h % 3].at[pl.ds(0, n)]

    def stage(h):
        @pl.when(sid == h % 2)
        def _():
            pltpu.async_copy(src(h), dst(h), rsems[h % 3])

    def wait_stage(h):
        @pl.when(sid == h % 2)
        def _():
            pltpu.make_async_copy(src(h), dst(h), rsems[h % 3]).wait()

    stage(0)
    stage(1)
    ocp = [None, None]
    for h in range(NH):
        row_k, part = h // 2, h % 2
        wait_stage(h)
        plsc.subcore_barrier()
        gl_v = ga_v if part == 0 else gb_v
        gcp = pltpu.async_copy(row_sh[h % 3].at[idxa_v], gl_v, gsem)
        if h + 2 < NH:
            stage(h + 2)
        gcp.wait()
        if part == 1:
            if ocp[row_k % 2] is not None:
                ocp[row_k % 2].wait()

            def merge(k, carry):
                v = idx_v[pl.ds(k * L, L)]
                a = ga_v[pl.ds(k * L, L)]
                b = gb_v[pl.ds(k * L, L)]
                out_v[row_k % 2, pl.ds(k * L, L)] = jnp.where(v < W0, a, b)
                return carry

            lax.fori_loop(0, SEG // L, merge, 0)
            r = cid * RPC + row_k
            ocp[row_k % 2] = pltpu.async_copy(
                out_v.at[row_k % 2],
                out_hbm.at[r].at[pl.ds(sid * SEG, SEG)], osem)
    for o in ocp:
        o.wait()


def kernel(data, indices):
    mesh = plsc.VectorSubcoreMesh(core_axis_name="c", subcore_axis_name="s")
    gather = pl.kernel(
        _body,
        mesh=mesh,
        out_type=jax.ShapeDtypeStruct((R, B), jnp.float32),
        scratch_types=[
            pltpu.VMEM((SEG,), jnp.int32),
            pltpu.VMEM((SEG,), jnp.int32),
            pltpu.VMEM((SEG,), jnp.float32),
            pltpu.VMEM((SEG,), jnp.float32),
            pltpu.VMEM((2, SEG), jnp.float32),
            pltpu.VMEM_SHARED((W1,), jnp.float32),
            pltpu.VMEM_SHARED((W1,), jnp.float32),
            pltpu.VMEM_SHARED((W1,), jnp.float32),
            pltpu.SemaphoreType.DMA,
            pltpu.SemaphoreType.DMA,
            pltpu.SemaphoreType.DMA,
            pltpu.SemaphoreType.DMA,
            pltpu.SemaphoreType.DMA,
        ],
    )
    return gather(data, indices)


# speedup vs baseline: 7.6585x; 1.0062x over previous
import jax
import jax.numpy as jnp
from jax import lax
from jax.experimental import pallas as pl
from jax.experimental.pallas import tpu as pltpu
from jax.experimental.pallas import tpu_sc as plsc

R, V, B = 64, 1_000_000, 16384
NC, NS = 2, 16
RPC = R // NC
SEG = B // NS
QW = 249984
QLEN = (QW, QW, QW, V - 3 * QW)
NQ = 4
NH = RPC * NQ
RING = 6
L = 16


def _body(data_hbm, idx_hbm, out_hbm,
          idx_v, il_v, q_v, g0_v, g1_v, g2_v, g3_v, out_v,
          sh0, sh1, sh2, sh3, sh4, sh5,
          rsem0, rsem1, rsem2, rsem3, rsem4, rsem5, gsem, osem):
    cid = lax.axis_index("c")
    sid = lax.axis_index("s")
    row_sh = (sh0, sh1, sh2, sh3, sh4, sh5)
    rsems = (rsem0, rsem1, rsem2, rsem3, rsem4, rsem5)
    g_v = (g0_v, g1_v, g2_v, g3_v)
    pltpu.sync_copy(idx_hbm.at[pl.ds(sid * SEG, SEG)], idx_v)

    def prep(k, carry):
        v = idx_v[pl.ds(k * L, L)]
        one = jnp.int32(1)
        zero = jnp.int32(0)
        q = (jnp.where(v >= QW, one, zero)
             + jnp.where(v >= 2 * QW, one, zero)
             + jnp.where(v >= 3 * QW, one, zero))
        q_v[pl.ds(k * L, L)] = q
        il_v[pl.ds(k * L, L)] = v - q * QW
        return carry

    lax.fori_loop(0, SEG // L, prep, 0)

    def parts(h):
        part = h % NQ
        r = cid * RPC + h // NQ
        src = data_hbm.at[r].at[pl.ds(part * QW, QLEN[part])]
        dst = row_sh[h % RING].at[pl.ds(0, QLEN[part])]
        return src, dst

    def stage(h):
        src, dst = parts(h)

        @pl.when(sid == h % RING)
        def _():
            pltpu.async_copy(src, dst, rsems[h % RING])

    def wait_stage(h):
        src, dst = parts(h)

        @pl.when(sid == h % RING)
        def _():
            pltpu.make_async_copy(src, dst, rsems[h % RING]).wait()

    for h in range(RING - 1):
        stage(h)
    ocp = [None, None]
    for h in range(NH):
        row_k, part = h // NQ, h % NQ
        wait_stage(h)
        plsc.subcore_barrier()
        gcp = pltpu.async_copy(row_sh[h % RING].at[il_v], g_v[part], gsem)
        if h + RING - 1 < NH:
            stage(h + RING - 1)
        gcp.wait()
        if part == NQ - 1:
            if ocp[row_k % 2] is not None:
                ocp[row_k % 2].wait()

            def merge(k, carry):
                q = q_v[pl.ds(k * L, L)]
                a = g0_v[pl.ds(k * L, L)]
                b = g1_v[pl.ds(k * L, L)]
                c = g2_v[pl.ds(k * L, L)]
                d = g3_v[pl.ds(k * L, L)]
                out_v[row_k % 2, pl.ds(k * L, L)] = jnp.where(
                    q < 2, jnp.where(q == 0, a, b), jnp.where(q == 2, c, d))
                return carry

            lax.fori_loop(0, SEG // L, merge, 0)
            r = cid * RPC + row_k
            ocp[row_k % 2] = pltpu.async_copy(
                out_v.at[row_k % 2],
                out_hbm.at[r].at[pl.ds(sid * SEG, SEG)], osem)
    for o in ocp:
        o.wait()


def kernel(data, indices):
    mesh = plsc.VectorSubcoreMesh(core_axis_name="c", subcore_axis_name="s")
    slot = pltpu.VMEM_SHARED((QLEN[-1],), jnp.float32)
    gather = pl.kernel(
        _body,
        mesh=mesh,
        out_type=jax.ShapeDtypeStruct((R, B), jnp.float32),
        scratch_types=[
            pltpu.VMEM((SEG,), jnp.int32),
            pltpu.VMEM((SEG,), jnp.int32),
            pltpu.VMEM((SEG,), jnp.int32),
            pltpu.VMEM((SEG,), jnp.float32),
            pltpu.VMEM((SEG,), jnp.float32),
            pltpu.VMEM((SEG,), jnp.float32),
            pltpu.VMEM((SEG,), jnp.float32),
            pltpu.VMEM((2, SEG), jnp.float32),
            slot, slot, slot, slot, slot, slot,
            pltpu.SemaphoreType.DMA,
            pltpu.SemaphoreType.DMA,
            pltpu.SemaphoreType.DMA,
            pltpu.SemaphoreType.DMA,
            pltpu.SemaphoreType.DMA,
            pltpu.SemaphoreType.DMA,
            pltpu.SemaphoreType.DMA,
            pltpu.SemaphoreType.DMA,
        ],
    )
    return gather(data, indices)
